# baseline (speedup 1.0000x reference)
import functools

import jax
import jax.numpy as jnp
from jax import lax
from jax.experimental import pallas as pl
from jax.experimental.pallas import tpu as pltpu
from jax.experimental.pallas import tpu_sc as plsc

BSZ, SEQ, IN_DIM, PROJ_DIM, Q = 4, 256, 768, 128, 16384
ROWS = BSZ * SEQ
TBL_ROWS = ROWS + 128
ZERO_ROW = ROWS

NC, NS = 2, 16
NW = NC * NS
QPW = Q // NW
QPI = 64
ISSUES = QPW // QPI
IDX_N = 2 * QPI


def _proj_body(x_ref, w_ref, b_ref, o_ref):
    i = pl.program_id(0)

    @pl.when(i < ROWS // 128)
    def _():
        o_ref[...] = (
            jnp.dot(x_ref[...], w_ref[...], preferred_element_type=jnp.float32)
            + b_ref[...]
        )

    @pl.when(i >= ROWS // 128)
    def _():
        o_ref[...] = jnp.zeros_like(o_ref)


def _project(x2d, W, b2d):
    nblk = TBL_ROWS // 128
    return pl.pallas_call(
        _proj_body,
        grid=(nblk,),
        in_specs=[
            pl.BlockSpec((128, IN_DIM), lambda i: (jnp.minimum(i, ROWS // 128 - 1), 0)),
            pl.BlockSpec((IN_DIM, PROJ_DIM), lambda i: (0, 0)),
            pl.BlockSpec((1, PROJ_DIM), lambda i: (0, 0)),
        ],
        out_specs=pl.BlockSpec((128, PROJ_DIM), lambda i: (i, 0)),
        out_shape=jax.ShapeDtypeStruct((TBL_ROWS, PROJ_DIM), jnp.float32),
    )(x2d, W, b2d)


def _gather_body(table, s1, e1, qb, s2, e2, out1, out2,
                 s1v, e1v, qbv, s2v, e2v, idxv, rowsv, sem):
    wid = lax.axis_index("s") * NC + lax.axis_index("c")
    qbase = wid * QPW
    pltpu.sync_copy(s1.at[pl.ds(qbase, QPW)], s1v)
    pltpu.sync_copy(e1.at[pl.ds(qbase, QPW)], e1v)
    pltpu.sync_copy(qb.at[pl.ds(qbase, QPW)], qbv)
    pltpu.sync_copy(s2.at[pl.ds(qbase, QPW)], s2v)
    pltpu.sync_copy(e2.at[pl.ds(qbase, QPW)], e2v)

    for out_ref, sv, ev in ((out1, s1v, e1v), (out2, s2v, e2v)):
        for j in range(ISSUES):
            for t in range(QPI // 16):
                qo = j * QPI + t * 16
                s = sv[pl.ds(qo, 16)]
                e = ev[pl.ds(qo, 16)]
                bb = qbv[pl.ds(qo, 16)]
                valid = e >= s
                fs = jnp.where(valid, bb * SEQ + s, ZERO_ROW)
                fe = jnp.where(valid, bb * SEQ + e, ZERO_ROW)
                idxv[pl.ds(t * 16, 16)] = fs
                idxv[pl.ds(QPI + t * 16, 16)] = fe
            pltpu.async_copy(table.at[idxv], rowsv, sem).wait()
            qrow = qbase + j * QPI
            pltpu.sync_copy(
                rowsv.at[pl.ds(0, QPI)],
                out_ref.at[pl.ds(qrow, QPI), pl.ds(0, PROJ_DIM)],
            )
            pltpu.sync_copy(
                rowsv.at[pl.ds(QPI, QPI)],
                out_ref.at[pl.ds(qrow, QPI), pl.ds(PROJ_DIM, PROJ_DIM)],
            )


def _span_gather_sc(table, s1, e1, qb, s2, e2):
    mesh = plsc.VectorSubcoreMesh(
        core_axis_name="c", subcore_axis_name="s", num_cores=NC, num_subcores=NS
    )
    f = functools.partial(
        pl.kernel,
        out_type=(
            jax.ShapeDtypeStruct((Q, 2 * PROJ_DIM), jnp.float32),
            jax.ShapeDtypeStruct((Q, 2 * PROJ_DIM), jnp.float32),
        ),
        mesh=mesh,
        scratch_types=[
            pltpu.VMEM((QPW,), jnp.int32),
            pltpu.VMEM((QPW,), jnp.int32),
            pltpu.VMEM((QPW,), jnp.int32),
            pltpu.VMEM((QPW,), jnp.int32),
            pltpu.VMEM((QPW,), jnp.int32),
            pltpu.VMEM((IDX_N,), jnp.int32),
            pltpu.VMEM((IDX_N, PROJ_DIM), jnp.float32),
            pltpu.SemaphoreType.DMA,
        ],
    )(_gather_body)
    return f(table, s1, e1, qb, s2, e2)


def kernel(flag, encoded_input, start_ids_1, end_ids_1, query_batch_idx,
           start_ids_2, end_ids_2, W, b):
    x2d = encoded_input.reshape(ROWS, IN_DIM)
    table = _project(x2d, W, b.reshape(1, PROJ_DIM))
    s1 = start_ids_1.astype(jnp.int32)
    e1 = end_ids_1.astype(jnp.int32)
    qb = query_batch_idx.astype(jnp.int32)
    s2 = start_ids_2.astype(jnp.int32)
    e2 = end_ids_2.astype(jnp.int32)
    o1, o2 = _span_gather_sc(table, s1, e1, qb, s2, e2)
    return (o1, o2)
